# Initial kernel scaffold; baseline (speedup 1.0000x reference)
import jax
import jax.numpy as jnp
from jax.experimental import pallas as pl


def kernel(x, period, price, edge_index, emb, value, lstm_Wih, lstm_Whh, lstm_bih, lstm_bhh, convW2, convb2, convW3, convb3, convW4, convb4, fcW, fcb, fc1W, fc1b, gc1W, gc1b, gc2W, gc2b, gatW, attn_l, attn_r, gatb, outW, outb):
    raise NotImplementedError("write your pallas kernel here")


# speedup vs baseline: 1.0499x; 1.0499x over previous
import functools
import jax
import jax.numpy as jnp
from jax import lax
from jax.experimental import pallas as pl
from jax.experimental.pallas import tpu as pltpu

_H = 128
_SEQ = 20
_EMB = 256
_NF = 128
_HEADS = 4


def _enc_body(xe_ref, pinv_ref, wihT_ref, whhT_ref, bias_ref,
              cw2_ref, cw3_ref, cw4_ref, cb_ref,
              fcW_ref, fcb_ref, fc1W_ref, fc1b_ref,
              out_ref, hsf_ref, hsb_ref, hseq_ref):
    B = xe_ref.shape[1]
    f32 = jnp.float32
    for l in range(2):
        in_ref = xe_ref if l == 0 else hseq_ref
        for d in range(2):
            wihT = wihT_ref[l, d]
            whhT = whhT_ref[l, d]
            bias = bias_ref[l, d]
            hs_ref = hsf_ref if d == 0 else hsb_ref

            def step(i, carry, in_ref=in_ref, wihT=wihT, whhT=whhT,
                     bias=bias, hs_ref=hs_ref, d=d):
                h, c = carry
                t = i if d == 0 else (_SEQ - 1 - i)
                xt = in_ref[t]
                g = (jnp.dot(xt, wihT, preferred_element_type=f32)
                     + jnp.dot(h, whhT, preferred_element_type=f32) + bias)
                gi = jax.nn.sigmoid(g[:, 0 * _H:1 * _H])
                gf = jax.nn.sigmoid(g[:, 1 * _H:2 * _H])
                gg = jnp.tanh(g[:, 2 * _H:3 * _H])
                go = jax.nn.sigmoid(g[:, 3 * _H:4 * _H])
                c = gf * c + gi * gg
                h = go * jnp.tanh(c)
                hs_ref[t] = h
                return (h, c)

            z = jnp.zeros((B, _H), f32)
            lax.fori_loop(0, _SEQ, step, (z, z))
        hseq_ref[...] = jnp.concatenate([hsf_ref[...], hsb_ref[...]], axis=-1)

    hseq2 = hseq_ref[...].reshape(_SEQ * B, 2 * _H)
    feats = []
    for ci, (cw_ref, k) in enumerate(((cw2_ref, 2), (cw3_ref, 3), (cw4_ref, 4))):
        Ps = [jnp.dot(hseq2, cw_ref[j], preferred_element_type=f32)
              .reshape(_SEQ, B, _NF) for j in range(k)]
        m = None
        for t in range(_SEQ - k + 1):
            y = Ps[0][t]
            for j in range(1, k):
                y = y + Ps[j][t + j]
            m = y if m is None else jnp.maximum(m, y)
        feats.append(jax.nn.relu(m + cb_ref[ci]))
    o = jnp.concatenate(feats, axis=-1)
    o = jnp.tanh(jnp.dot(o, fcW_ref[...], preferred_element_type=f32) + fcb_ref[...])
    o = o * pinv_ref[...]
    o = jnp.tanh(jnp.dot(o, fc1W_ref[...], preferred_element_type=f32) + fc1b_ref[...])
    out_ref[...] = o


def _encoder(xeT, pinv, wihT, whhT, bias, cw2, cw3, cw4, cb,
             fcW, fcb, fc1W, fc1b, block=128):
    NP = xeT.shape[1]
    grid = (NP // block,)
    c0 = lambda i: (0, 0, 0)
    return pl.pallas_call(
        _enc_body,
        grid=grid,
        in_specs=[
            pl.BlockSpec((_SEQ, block, _EMB), lambda i: (0, i, 0)),
            pl.BlockSpec((block, 1), lambda i: (i, 0)),
            pl.BlockSpec((2, 2, _EMB, 4 * _H), lambda i: (0, 0, 0, 0)),
            pl.BlockSpec((2, 2, _H, 4 * _H), lambda i: (0, 0, 0, 0)),
            pl.BlockSpec((2, 2, 4 * _H), c0),
            pl.BlockSpec((2, _EMB, _NF), c0),
            pl.BlockSpec((3, _EMB, _NF), c0),
            pl.BlockSpec((4, _EMB, _NF), c0),
            pl.BlockSpec((3, _NF), lambda i: (0, 0)),
            pl.BlockSpec((3 * _NF, _H), lambda i: (0, 0)),
            pl.BlockSpec((1, _H), lambda i: (0, 0)),
            pl.BlockSpec((_H, _H), lambda i: (0, 0)),
            pl.BlockSpec((1, _H), lambda i: (0, 0)),
        ],
        out_specs=pl.BlockSpec((block, _H), lambda i: (i, 0)),
        out_shape=jax.ShapeDtypeStruct((NP, _H), jnp.float32),
        scratch_shapes=[
            pltpu.VMEM((_SEQ, block, _H), jnp.float32),
            pltpu.VMEM((_SEQ, block, _H), jnp.float32),
            pltpu.VMEM((_SEQ, block, 2 * _H), jnp.float32),
        ],
    )(xeT, pinv, wihT, whhT, bias, cw2, cw3, cw4, cb, fcW, fcb, fc1W, fc1b)


def _graph(o, price, edge_index, gc1W, gc1b, gc2W, gc2b, gatW, attn_l,
           attn_r, gatb, outW, outb):
    N = o.shape[0]
    o = jnp.concatenate([o, price], axis=1)
    src = jnp.concatenate([edge_index[0], jnp.arange(N)])
    dst = jnp.concatenate([edge_index[1], jnp.arange(N)])
    norm_src = lax.rsqrt(jnp.maximum(jnp.bincount(src, length=N).astype(jnp.float32), 1.0))
    norm_dst = lax.rsqrt(jnp.maximum(jnp.bincount(dst, length=N).astype(jnp.float32), 1.0))

    def gcn(hh, W, b):
        m = (hh * norm_src[:, None])[src]
        agg = jax.ops.segment_sum(m, dst, num_segments=N)
        return (agg * norm_dst[:, None]) @ W + b

    o = jnp.tanh(gcn(o, gc1W, gc1b))
    o = jnp.tanh(gcn(o, gc2W, gc2b))
    feat = (o @ gatW).reshape(N, _HEADS, _H)
    el = jnp.sum(feat * attn_l[None], axis=-1)
    er = jnp.sum(feat * attn_r[None], axis=-1)
    e = jax.nn.leaky_relu(el[src] + er[dst], 0.2)
    emax = jax.ops.segment_max(e, dst, num_segments=N)
    ex = jnp.exp(e - emax[dst])
    esum = jax.ops.segment_sum(ex, dst, num_segments=N)
    alpha = ex / jnp.maximum(esum[dst], 1e-9)
    rst = jax.ops.segment_sum(feat[src] * alpha[:, :, None], dst, num_segments=N)
    rst = rst + gatb.reshape(_HEADS, _H)[None]
    o = jnp.tanh(jnp.mean(rst, axis=1))
    return o @ outW + outb


def kernel(x, period, price, edge_index, emb, value, lstm_Wih, lstm_Whh,
           lstm_bih, lstm_bhh, convW2, convb2, convW3, convb3, convW4,
           convb4, fcW, fcb, fc1W, fc1b, gc1W, gc1b, gc2W, gc2b, gatW,
           attn_l, attn_r, gatb, outW, outb):
    N = x.shape[0]
    block = 128
    NP = ((N + block - 1) // block) * block

    h0 = emb[x]
    xeT = jnp.swapaxes(h0, 0, 1)
    xeT = jnp.pad(xeT, ((0, 0), (0, NP - N), (0, 0)))
    pinv = jnp.exp(-value * period / 30.0)[:, None]
    pinv = jnp.pad(pinv, ((0, NP - N), (0, 0)))

    wihT = jnp.swapaxes(lstm_Wih, 2, 3)
    whhT = jnp.swapaxes(lstm_Whh, 2, 3)
    bias = lstm_bih + lstm_bhh
    cw2 = jnp.transpose(convW2[:, 0], (1, 2, 0))
    cw3 = jnp.transpose(convW3[:, 0], (1, 2, 0))
    cw4 = jnp.transpose(convW4[:, 0], (1, 2, 0))
    cb = jnp.stack([convb2, convb3, convb4])

    o = _encoder(xeT, pinv, wihT, whhT, bias, cw2, cw3, cw4, cb,
                 fcW, fcb[None], fc1W, fc1b[None], block=block)[:N]

    return _graph(o, price, edge_index, gc1W, gc1b, gc2W, gc2b, gatW,
                  attn_l, attn_r, gatb, outW, outb)


# speedup vs baseline: 1.1480x; 1.0933x over previous
import functools
import jax
import jax.numpy as jnp
from jax import lax
from jax.experimental import pallas as pl
from jax.experimental.pallas import tpu as pltpu
from jax.experimental.pallas import tpu_sc as plsc

_H = 128
_SEQ = 20
_EMB = 256
_NF = 128
_HEADS = 4

_NCORE = 2
_NSUB = 16
_NW = _NCORE * _NSUB


def _sc_mesh():
    return plsc.VectorSubcoreMesh(core_axis_name="c", subcore_axis_name="s")


def _chunks(per_w, cmax=128):
    out = []
    nfull = per_w // cmax
    if nfull:
        out.append((cmax, nfull))
    rem = per_w - nfull * cmax
    if rem:
        assert rem % 8 == 0, rem
        out.append((rem, 1))
    return out


def _seg_sum_sc(feat, src, dst, N):
    E = src.shape[0]
    D = feat.shape[1]
    per_w = E // _NW
    assert per_w * _NW == E and (D * 4) % 64 == 0
    Np = ((N + 127) // 128) * 128
    rows_pw = Np // _NSUB
    C = 128
    pieces = _chunks(per_w, C)

    idx_scratch = []
    for (csz, _) in pieces:
        idx_scratch += [pltpu.VMEM((csz,), jnp.int32),
                        pltpu.VMEM((csz,), jnp.int32)]

    @functools.partial(
        pl.kernel, mesh=_sc_mesh(),
        out_type=jax.ShapeDtypeStruct((2 * Np, D), jnp.float32),
        scratch_types=[
            pltpu.VMEM((C, D), jnp.float32),
            pltpu.VMEM_SHARED((Np, D), jnp.float32),
            pltpu.SemaphoreType.DMA,
        ] + idx_scratch,
    )
    def k(feat_hbm, src_hbm, dst_hbm, zeros_hbm, out_hbm,
          rows_v, acc_sh, sem, *idx_vs):
        ci = lax.axis_index("c")
        si = lax.axis_index("s")
        wid = si * _NCORE + ci
        pltpu.sync_copy(zeros_hbm, acc_sh.at[pl.ds(si * rows_pw, rows_pw)])
        plsc.subcore_barrier()
        base = wid * per_w
        off = 0
        for pi, (csz, cnt) in enumerate(pieces):
            sidx_v = idx_vs[2 * pi]
            didx_v = idx_vs[2 * pi + 1]

            def body(i, _, off=off, csz=csz, sidx_v=sidx_v, didx_v=didx_v):
                b = base + off + i * csz
                pltpu.sync_copy(src_hbm.at[pl.ds(b, csz)], sidx_v)
                pltpu.sync_copy(dst_hbm.at[pl.ds(b, csz)], didx_v)
                pltpu.async_copy(feat_hbm.at[sidx_v],
                                 rows_v.at[pl.ds(0, csz)], sem).wait()
                pltpu.sync_copy(rows_v.at[pl.ds(0, csz)],
                                acc_sh.at[didx_v], add=True)
                return 0
            lax.fori_loop(0, cnt, body, 0)
            off += csz * cnt
        plsc.subcore_barrier()
        r0 = si * rows_pw
        pltpu.sync_copy(acc_sh.at[pl.ds(r0, rows_pw)],
                        out_hbm.at[pl.ds(ci * Np + r0, rows_pw)])

    zeros = jnp.zeros((rows_pw, D), jnp.float32)
    out = k(feat, src, dst, zeros).reshape(2, Np, D)
    return out[:, :N]


def _enc_body(xe_ref, pinv_ref, wihT_ref, whhT_ref, bias_ref,
              cw2_ref, cw3_ref, cw4_ref, cb_ref,
              fcW_ref, fcb_ref, fc1W_ref, fc1b_ref,
              out_ref, hsf_ref, hsb_ref, hseq_ref):
    B = xe_ref.shape[1]
    f32 = jnp.float32
    for l in range(2):
        in_ref = xe_ref if l == 0 else hseq_ref
        for d in range(2):
            wihT = wihT_ref[l, d]
            whhT = whhT_ref[l, d]
            bias = bias_ref[l, d]
            hs_ref = hsf_ref if d == 0 else hsb_ref

            def step(i, carry, in_ref=in_ref, wihT=wihT, whhT=whhT,
                     bias=bias, hs_ref=hs_ref, d=d):
                h, c = carry
                t = i if d == 0 else (_SEQ - 1 - i)
                xt = in_ref[t]
                g = (jnp.dot(xt, wihT, preferred_element_type=f32)
                     + jnp.dot(h, whhT, preferred_element_type=f32) + bias)
                gi = jax.nn.sigmoid(g[:, 0 * _H:1 * _H])
                gf = jax.nn.sigmoid(g[:, 1 * _H:2 * _H])
                gg = jnp.tanh(g[:, 2 * _H:3 * _H])
                go = jax.nn.sigmoid(g[:, 3 * _H:4 * _H])
                c = gf * c + gi * gg
                h = go * jnp.tanh(c)
                hs_ref[t] = h
                return (h, c)

            z = jnp.zeros((B, _H), f32)
            lax.fori_loop(0, _SEQ, step, (z, z))
        hseq_ref[...] = jnp.concatenate([hsf_ref[...], hsb_ref[...]], axis=-1)

    hseq2 = hseq_ref[...].reshape(_SEQ * B, 2 * _H)
    feats = []
    for ci, (cw_ref, k) in enumerate(((cw2_ref, 2), (cw3_ref, 3), (cw4_ref, 4))):
        Ps = [jnp.dot(hseq2, cw_ref[j], preferred_element_type=f32)
              .reshape(_SEQ, B, _NF) for j in range(k)]
        m = None
        for t in range(_SEQ - k + 1):
            y = Ps[0][t]
            for j in range(1, k):
                y = y + Ps[j][t + j]
            m = y if m is None else jnp.maximum(m, y)
        feats.append(jax.nn.relu(m + cb_ref[ci]))
    o = jnp.concatenate(feats, axis=-1)
    o = jnp.tanh(jnp.dot(o, fcW_ref[...], preferred_element_type=f32) + fcb_ref[...])
    o = o * pinv_ref[...]
    o = jnp.tanh(jnp.dot(o, fc1W_ref[...], preferred_element_type=f32) + fc1b_ref[...])
    out_ref[...] = o


def _encoder(xeT, pinv, wihT, whhT, bias, cw2, cw3, cw4, cb,
             fcW, fcb, fc1W, fc1b, block=128):
    NP = xeT.shape[1]
    grid = (NP // block,)
    c0 = lambda i: (0, 0, 0)
    return pl.pallas_call(
        _enc_body,
        grid=grid,
        in_specs=[
            pl.BlockSpec((_SEQ, block, _EMB), lambda i: (0, i, 0)),
            pl.BlockSpec((block, 1), lambda i: (i, 0)),
            pl.BlockSpec((2, 2, _EMB, 4 * _H), lambda i: (0, 0, 0, 0)),
            pl.BlockSpec((2, 2, _H, 4 * _H), lambda i: (0, 0, 0, 0)),
            pl.BlockSpec((2, 2, 4 * _H), c0),
            pl.BlockSpec((2, _EMB, _NF), c0),
            pl.BlockSpec((3, _EMB, _NF), c0),
            pl.BlockSpec((4, _EMB, _NF), c0),
            pl.BlockSpec((3, _NF), lambda i: (0, 0)),
            pl.BlockSpec((3 * _NF, _H), lambda i: (0, 0)),
            pl.BlockSpec((1, _H), lambda i: (0, 0)),
            pl.BlockSpec((_H, _H), lambda i: (0, 0)),
            pl.BlockSpec((1, _H), lambda i: (0, 0)),
        ],
        out_specs=pl.BlockSpec((block, _H), lambda i: (i, 0)),
        out_shape=jax.ShapeDtypeStruct((NP, _H), jnp.float32),
        scratch_shapes=[
            pltpu.VMEM((_SEQ, block, _H), jnp.float32),
            pltpu.VMEM((_SEQ, block, _H), jnp.float32),
            pltpu.VMEM((_SEQ, block, 2 * _H), jnp.float32),
        ],
    )(xeT, pinv, wihT, whhT, bias, cw2, cw3, cw4, cb, fcW, fcb, fc1W, fc1b)


def _graph(o, price, edge_index, gc1W, gc1b, gc2W, gc2b, gatW, attn_l,
           attn_r, gatb, outW, outb):
    N = o.shape[0]
    o = jnp.concatenate([o, price], axis=1)
    esrc = edge_index[0]
    edst = edge_index[1]
    ones = jnp.ones((N, 128), jnp.float32)
    cnt_src = _seg_sum_sc(ones, esrc, esrc, N).sum(0)[:, 0]
    cnt_dst = _seg_sum_sc(ones, edst, edst, N).sum(0)[:, 0]
    norm_src = lax.rsqrt(cnt_src + 1.0)
    norm_dst = lax.rsqrt(cnt_dst + 1.0)

    def gcn(hh, W, b):
        hs = (hh * norm_src[:, None]) @ W
        parts = _seg_sum_sc(hs, esrc, edst, N)
        agg = parts[0] + parts[1] + hs
        return agg * norm_dst[:, None] + b

    o = jnp.tanh(gcn(o, gc1W, gc1b))
    o = jnp.tanh(gcn(o, gc2W, gc2b))
    src = jnp.concatenate([esrc, jnp.arange(N)])
    dst = jnp.concatenate([edst, jnp.arange(N)])
    feat = (o @ gatW).reshape(N, _HEADS, _H)
    el = jnp.sum(feat * attn_l[None], axis=-1)
    er = jnp.sum(feat * attn_r[None], axis=-1)
    e = jax.nn.leaky_relu(el[src] + er[dst], 0.2)
    emax = jax.ops.segment_max(e, dst, num_segments=N)
    ex = jnp.exp(e - emax[dst])
    esum = jax.ops.segment_sum(ex, dst, num_segments=N)
    alpha = ex / jnp.maximum(esum[dst], 1e-9)
    rst = jax.ops.segment_sum(feat[src] * alpha[:, :, None], dst, num_segments=N)
    rst = rst + gatb.reshape(_HEADS, _H)[None]
    o = jnp.tanh(jnp.mean(rst, axis=1))
    return o @ outW + outb


def kernel(x, period, price, edge_index, emb, value, lstm_Wih, lstm_Whh,
           lstm_bih, lstm_bhh, convW2, convb2, convW3, convb3, convW4,
           convb4, fcW, fcb, fc1W, fc1b, gc1W, gc1b, gc2W, gc2b, gatW,
           attn_l, attn_r, gatb, outW, outb):
    N = x.shape[0]
    block = 128
    NP = ((N + block - 1) // block) * block

    h0 = emb[x]
    xeT = jnp.swapaxes(h0, 0, 1)
    xeT = jnp.pad(xeT, ((0, 0), (0, NP - N), (0, 0)))
    pinv = jnp.exp(-value * period / 30.0)[:, None]
    pinv = jnp.pad(pinv, ((0, NP - N), (0, 0)))

    wihT = jnp.swapaxes(lstm_Wih, 2, 3)
    whhT = jnp.swapaxes(lstm_Whh, 2, 3)
    bias = lstm_bih + lstm_bhh
    cw2 = jnp.transpose(convW2[:, 0], (1, 2, 0))
    cw3 = jnp.transpose(convW3[:, 0], (1, 2, 0))
    cw4 = jnp.transpose(convW4[:, 0], (1, 2, 0))
    cb = jnp.stack([convb2, convb3, convb4])

    o = _encoder(xeT, pinv, wihT, whhT, bias, cw2, cw3, cw4, cb,
                 fcW, fcb[None], fc1W, fc1b[None], block=block)[:N]

    return _graph(o, price, edge_index, gc1W, gc1b, gc2W, gc2b, gatW,
                  attn_l, attn_r, gatb, outW, outb)
